# Initial kernel scaffold; baseline (speedup 1.0000x reference)
import jax
import jax.numpy as jnp
from jax.experimental import pallas as pl


def kernel(data, params, channel_transform, spatio_transform):
    raise NotImplementedError("write your pallas kernel here")


# speedup vs baseline: 2619.6516x; 2619.6516x over previous
import functools

import jax
import jax.numpy as jnp
import numpy as np
from jax import lax
from jax.experimental import pallas as pl
from jax.experimental.pallas import tpu as pltpu
from jax.experimental.pallas import tpu_sc as plsc

_POINTS = 120
_ROWS = 4704
_COLS = 4096
_N = _ROWS * _COLS
_RB = 96
_GRID = _ROWS // _RB

_NW = 32
_WPW = _N // _NW
_CHUNK = 12288
_NCHUNK = _WPW // _CHUNK

_EPS_SP = float(np.spacing(np.finfo(np.float32).eps))
_FINE = 128


def _minmax_body(x_ref, out_ref, acc_ref):
    i = pl.program_id(0)
    x = x_ref[...]
    bmn = jnp.min(x, axis=0, keepdims=True)
    bmx = jnp.max(x, axis=0, keepdims=True)

    @pl.when(i == 0)
    def _():
        acc_ref[0:1, :] = bmn
        acc_ref[1:2, :] = bmx

    @pl.when(i > 0)
    def _():
        acc_ref[0:1, :] = jnp.minimum(acc_ref[0:1, :], bmn)
        acc_ref[1:2, :] = jnp.maximum(acc_ref[1:2, :], bmx)

    @pl.when(i == _GRID - 1)
    def _():
        gmn = jnp.min(acc_ref[0:1, :])
        gmx = jnp.max(acc_ref[1:2, :])
        out_ref[0:1, :] = jnp.full((1, 128), gmn, jnp.float32)
        out_ref[1:2, :] = jnp.full((1, 128), gmx, jnp.float32)


def _minmax(x2d):
    return pl.pallas_call(
        _minmax_body,
        grid=(_GRID,),
        in_specs=[pl.BlockSpec((_RB, _COLS), lambda i: (i, 0))],
        out_specs=pl.BlockSpec((2, 128), lambda i: (0, 0)),
        out_shape=jax.ShapeDtypeStruct((2, 128), jnp.float32),
        scratch_shapes=[pltpu.VMEM((2, _COLS), jnp.float32)],
    )(x2d)


def _sc_hist_body(data_hbm, scal_hbm, out_hbm, buf0, buf1, hist, scal_v, orow, sem0, sem1):
    wid = lax.axis_index("c") * 16 + lax.axis_index("s")
    base = wid * _WPW

    pltpu.sync_copy(scal_hbm, scal_v)
    cvec = scal_v[pl.ds(0, 16)]
    dminv = scal_v[pl.ds(16, 16)]
    invwv = scal_v[pl.ds(32, 16)]

    zero16 = jnp.zeros((16,), jnp.int32)
    for h in range(128):
        hist[pl.ds(h * 16, 16)] = zero16

    lane_base = lax.iota(jnp.int32, 16) * 128
    ones16 = jnp.ones((16,), jnp.int32)

    bufs = (buf0, buf1)
    sems = (sem0, sem1)
    cps = [None, None]
    cps[0] = pltpu.async_copy(data_hbm.at[pl.ds(base, _CHUNK)], buf0, sem0)

    def _mk_inner(buf):
        def inner(v, carry):
            x = buf[pl.ds(v * 16, 16)]
            u = (x * cvec - dminv) * invwv
            bi = jnp.clip(u.astype(jnp.int32), 0, _POINTS - 1)
            plsc.addupdate_scatter(hist, [bi + lane_base], ones16)
            return carry
        return inner

    for ci in range(_NCHUNK):
        b = ci % 2
        cps[b].wait()
        if ci + 1 < _NCHUNK:
            nb = (ci + 1) % 2
            cps[nb] = pltpu.async_copy(
                data_hbm.at[pl.ds(base + (ci + 1) * _CHUNK, _CHUNK)],
                bufs[nb],
                sems[nb],
            )
        lax.fori_loop(0, _CHUNK // 16, _mk_inner(bufs[b]), 0)

    for j in range(8):
        acc = hist[pl.ds(j * 16, 16)]
        for r in range(1, 16):
            acc = acc + hist[pl.ds(r * 128 + j * 16, 16)]
        orow[pl.ds(j * 16, 16)] = acc

    pltpu.sync_copy(orow, out_hbm.at[wid])


@functools.cache
def _get_sc_hist():
    mesh = plsc.VectorSubcoreMesh(
        core_axis_name="c", subcore_axis_name="s", num_cores=2, num_subcores=16
    )
    return pl.kernel(
        _sc_hist_body,
        out_type=jax.ShapeDtypeStruct((_NW, 128), jnp.int32),
        mesh=mesh,
        compiler_params=pltpu.CompilerParams(needs_layout_passes=False),
        scratch_types=[
            pltpu.VMEM((_CHUNK,), jnp.float32),
            pltpu.VMEM((_CHUNK,), jnp.float32),
            pltpu.VMEM((16 * 128,), jnp.int32),
            pltpu.VMEM((48,), jnp.float32),
            pltpu.VMEM((128,), jnp.int32),
            pltpu.SemaphoreType.DMA,
            pltpu.SemaphoreType.DMA,
        ],
    )


def _sc_hist(data1d, scal48):
    return _get_sc_hist()(data1d, scal48)


def _transform_body(x_ref, hist_ref, scal_ref, par_ref, out_ref, tab_ref):
    i = pl.program_id(0)

    c = scal_ref[0]
    dmin = scal_ref[1]
    invw = scal_ref[2]
    spatio = scal_ref[3]

    @pl.when(i == 0)
    def _():
        hist_i = hist_ref[...]
        hcnt = jnp.sum(hist_i, axis=0, keepdims=True)
        r2 = lax.broadcasted_iota(jnp.int32, (128, 128), 0)
        l2 = lax.broadcasted_iota(jnp.int32, (128, 128), 1)
        hcnt_b = jnp.broadcast_to(hcnt, (128, 128))
        cum_col = jnp.sum(
            jnp.where((l2 <= r2) & (l2 < _POINTS), hcnt_b, 0),
            axis=1, keepdims=True)
        total = jnp.max(cum_col).astype(jnp.float32)
        accum_col = cum_col.astype(jnp.float32) * ((_POINTS - 1) / total)

        acc_b = jnp.broadcast_to(accum_col, (128, 128))
        acc_row = jnp.sum(jnp.where(r2 == l2, acc_b, 0.0),
                          axis=0, keepdims=True)
        acc_row8 = jnp.broadcast_to(acc_row, (8, 128))

        io_row8 = lax.broadcasted_iota(jnp.int32, (8, 128), 1)
        idxp1 = jnp.minimum(io_row8 + 1, _POINTS - 1)
        acc1_row8 = jnp.take_along_axis(acc_row8, idxp1, axis=1)

        tf = io_row8.astype(jnp.float32)
        ssum = jnp.sum(
            jnp.where((acc_b <= l2.astype(jnp.float32)) & (r2 < _POINTS), 1, 0),
            axis=0, keepdims=True)
        it = jnp.broadcast_to(jnp.clip(ssum, 1, _POINTS - 1), (8, 128))
        a1 = jnp.take_along_axis(acc_row8, it, axis=1)
        a0 = jnp.take_along_axis(acc_row8, it - 1, axis=1)
        dxv = a1 - a0
        dx0 = jnp.abs(dxv) <= _EPS_SP
        safe = jnp.where(dx0, 1.0, dxv)
        accum0 = jnp.sum(jnp.where((r2 == 0) & (l2 == 0), acc_b, 0.0))
        accum_last = jnp.max(accum_col)

        pt_row8 = jnp.broadcast_to(par_ref[0:1, :], (8, 128))
        pv_row8 = jnp.broadcast_to(par_ref[1:2, :], (8, 128))
        io1_row = lax.broadcasted_iota(jnp.int32, (1, 128), 1)
        pt0_s = jnp.sum(jnp.where(io1_row == 0, par_ref[0:1, :], 0.0))
        ptl_s = jnp.sum(jnp.where(io1_row == _POINTS - 1, par_ref[0:1, :], 0.0))
        pv0_s = jnp.sum(jnp.where(io1_row == 0, par_ref[1:2, :], 0.0))
        pvl_s = jnp.sum(jnp.where(io1_row == _POINTS - 1, par_ref[1:2, :], 0.0))

        def frame(p_row8, p0_s, pl_s):
            p1 = jnp.take_along_axis(p_row8, it, axis=1)
            p0 = jnp.take_along_axis(p_row8, it - 1, axis=1)
            f = jnp.where(dx0, p0, p0 + (tf - a0) / safe * (p1 - p0))
            f = jnp.where(tf < accum0, p0_s, f)
            f = jnp.where(tf > accum_last, pl_s, f)
            return f

        fth = frame(pt_row8, pt0_s, ptl_s)
        fve = frame(pv_row8, pv0_s, pvl_s)
        fth1 = jnp.take_along_axis(fth, idxp1, axis=1)
        fve1 = jnp.take_along_axis(fve, idxp1, axis=1)

        def tak8(tab, idx):
            return jnp.take_along_axis(tab, idx, axis=1,
                                       mode="promise_in_bounds")

        def eval_exact(dk):
            ugk = (dk - dmin) * invw - 0.5
            jk = jnp.clip(ugk.astype(jnp.int32), 0, _POINTS - 2)
            tgk = jnp.clip(ugk - jk.astype(jnp.float32), 0.0, 1.0)
            av0 = tak8(acc_row8, jk)
            av1 = tak8(acc1_row8, jk)
            idx = av0 + tgk * (av1 - av0)
            bk = jnp.minimum(idx.astype(jnp.int32), _POINTS - 1)
            posk = idx - bk.astype(jnp.float32)
            thk = ((1.0 - posk) * tak8(fth, bk) + posk * tak8(fth1, bk))
            vek = ((1.0 - posk) * tak8(fve, bk) + posk * tak8(fve1, bk))
            dsk = vek * 0.01
            return dsk * jnp.sin(thk), dsk * jnp.cos(thk)

        cw = (1.0 / invw) * (float(_POINTS) / _FINE)
        kf2 = lax.broadcasted_iota(jnp.int32, (8, 128), 1).astype(jnp.float32)
        dy0, dx0 = eval_exact(dmin + kf2 * cw)
        dy1, dx1 = eval_exact(dmin + (kf2 + 1.0) * cw)

        tab_ref[0] = jnp.broadcast_to(dy0[0:1, :], (_RB, 128))
        tab_ref[1] = jnp.broadcast_to((dy1 - dy0)[0:1, :], (_RB, 128))
        tab_ref[2] = jnp.broadcast_to(dx0[0:1, :], (_RB, 128))
        tab_ref[3] = jnp.broadcast_to((dx1 - dx0)[0:1, :], (_RB, 128))

    def tak(tab, idx):
        return jnp.take_along_axis(tab, idx, axis=1, mode="promise_in_bounds")

    x = x_ref[...]
    d = x * c
    u = (d - dmin) * (invw * (_FINE / float(_POINTS)))
    k = jnp.minimum(u.astype(jnp.int32), _FINE - 1)
    frac = u - k.astype(jnp.float32)
    dyd = tak(tab_ref[0], k) + frac * tak(tab_ref[1], k)
    dxd = tak(tab_ref[2], k) + frac * tak(tab_ref[3], k)
    out_ref[...] = (d * (1.0 + dyd) + dxd) * spatio


def _transform(x2d, hist, scal, par):
    return pl.pallas_call(
        _transform_body,
        grid=(_GRID,),
        in_specs=[
            pl.BlockSpec((_RB, _COLS), lambda i: (i, 0)),
            pl.BlockSpec((_NW, 128), lambda i: (0, 0)),
            pl.BlockSpec(memory_space=pltpu.SMEM),
            pl.BlockSpec((2, 128), lambda i: (0, 0)),
        ],
        out_specs=pl.BlockSpec((_RB, _COLS), lambda i: (i, 0)),
        out_shape=jax.ShapeDtypeStruct((_ROWS, _COLS), jnp.float32),
        scratch_shapes=[pltpu.VMEM((4, _RB, 128), jnp.float32)],
    )(x2d, hist, scal, par)


def kernel(data, params, channel_transform, spatio_transform):
    shape = data.shape
    x2d = data.reshape(_ROWS, _COLS)

    mm = _minmax(x2d)
    mn = mm[0, 0]
    mx = mm[1, 0]

    c = channel_transform.reshape(())
    spatio = spatio_transform.reshape(())
    dmin = jnp.where(c >= 0, c * mn, c * mx)
    dmax = jnp.where(c >= 0, c * mx, c * mn)
    width = (dmax - dmin) / _POINTS
    invw = 1.0 / width

    scal48 = jnp.concatenate([
        jnp.full((16,), c, jnp.float32),
        jnp.full((16,), dmin, jnp.float32),
        jnp.full((16,), invw, jnp.float32),
    ])
    hist = _sc_hist(data.reshape(_N), scal48)

    scal4 = jnp.stack([c, dmin, invw, spatio]).astype(jnp.float32)
    par = jnp.pad(params.reshape(2, _POINTS), ((0, 0), (0, 128 - _POINTS)))
    out2d = _transform(x2d, hist, scal4, par)
    return out2d.reshape(shape)


# speedup vs baseline: 5674.5217x; 2.1661x over previous
import functools

import jax
import jax.numpy as jnp
import numpy as np
from jax import lax
from jax.experimental import pallas as pl
from jax.experimental.pallas import tpu as pltpu
from jax.experimental.pallas import tpu_sc as plsc

_POINTS = 120
_ROWS = 4704
_COLS = 4096
_N = _ROWS * _COLS
_RB = 96
_GRID = _ROWS // _RB

_NW = 32
_WPW = _N // _NW
_CHUNK = 12288
_NCHUNK = _WPW // _CHUNK

_EPS_SP = float(np.spacing(np.finfo(np.float32).eps))
_FINE = 128


def _minmax_body(x_ref, out_ref, acc_ref):
    i = pl.program_id(0)
    x = x_ref[...]
    bmn = jnp.min(x, axis=0, keepdims=True)
    bmx = jnp.max(x, axis=0, keepdims=True)

    @pl.when(i == 0)
    def _():
        acc_ref[0:1, :] = bmn
        acc_ref[1:2, :] = bmx

    @pl.when(i > 0)
    def _():
        acc_ref[0:1, :] = jnp.minimum(acc_ref[0:1, :], bmn)
        acc_ref[1:2, :] = jnp.maximum(acc_ref[1:2, :], bmx)

    @pl.when(i == _GRID - 1)
    def _():
        gmn = jnp.min(acc_ref[0:1, :])
        gmx = jnp.max(acc_ref[1:2, :])
        out_ref[0:1, :] = jnp.full((1, 128), gmn, jnp.float32)
        out_ref[1:2, :] = jnp.full((1, 128), gmx, jnp.float32)


def _minmax(x2d):
    return pl.pallas_call(
        _minmax_body,
        grid=(_GRID,),
        in_specs=[pl.BlockSpec((_RB, _COLS), lambda i: (i, 0))],
        out_specs=pl.BlockSpec((2, 128), lambda i: (0, 0)),
        out_shape=jax.ShapeDtypeStruct((2, 128), jnp.float32),
        scratch_shapes=[pltpu.VMEM((2, _COLS), jnp.float32)],
    )(x2d)


_UNROLL = 16


def _sc_hist_body(data_hbm, scal_hbm, out_hbm, buf0, buf1, hist, scal_v, orow, sem0, sem1):
    wid = lax.axis_index("c") * 16 + lax.axis_index("s")
    base = wid * _WPW

    pltpu.sync_copy(scal_hbm, scal_v)
    c2v = scal_v[pl.ds(0, 16)]
    d2v = scal_v[pl.ds(16, 16)]

    zero16 = jnp.zeros((16,), jnp.int32)
    for h in range(128):
        hist[pl.ds(h * 16, 16)] = zero16

    ones16 = jnp.ones((16,), jnp.int32)
    laneb_f = (lax.iota(jnp.int32, 16) * 128).astype(jnp.float32)
    d2s = d2v + laneb_f
    bndv = laneb_f + float(_POINTS - 1)

    def _compute(buf):
        def body(v):
            x = buf[pl.ds(v * 16, 16)]
            u = jnp.minimum(x * c2v + d2s, bndv)
            plsc.addupdate_scatter(hist, [u.astype(jnp.int32)], ones16)
        plsc.parallel_loop(0, _CHUNK // 16, 1, unroll=_UNROLL)(body)

    def _wait(buf, sem):
        pltpu.make_async_copy(data_hbm.at[pl.ds(0, _CHUNK)], buf, sem).wait()

    pltpu.async_copy(data_hbm.at[pl.ds(base, _CHUNK)], buf0, sem0)
    pltpu.async_copy(data_hbm.at[pl.ds(base + _CHUNK, _CHUNK)], buf1, sem1)

    def outer(ci, carry):
        k0 = 2 * ci
        _wait(buf0, sem0)
        _compute(buf0)

        @pl.when(k0 + 2 < _NCHUNK)
        def _():
            pltpu.async_copy(
                data_hbm.at[pl.ds(base + (k0 + 2) * _CHUNK, _CHUNK)], buf0, sem0)

        _wait(buf1, sem1)
        _compute(buf1)

        @pl.when(k0 + 3 < _NCHUNK)
        def _():
            pltpu.async_copy(
                data_hbm.at[pl.ds(base + (k0 + 3) * _CHUNK, _CHUNK)], buf1, sem1)

        return carry

    lax.fori_loop(0, _NCHUNK // 2, outer, 0)
    if _NCHUNK % 2:
        _wait(buf0, sem0)
        _compute(buf0)

    for j in range(8):
        acc = hist[pl.ds(j * 16, 16)]
        for r in range(1, 16):
            acc = acc + hist[pl.ds(r * 128 + j * 16, 16)]
        orow[pl.ds(j * 16, 16)] = acc

    pltpu.sync_copy(orow, out_hbm.at[wid])


@functools.cache
def _get_sc_hist():
    mesh = plsc.VectorSubcoreMesh(
        core_axis_name="c", subcore_axis_name="s", num_cores=2, num_subcores=16
    )
    return pl.kernel(
        _sc_hist_body,
        out_type=jax.ShapeDtypeStruct((_NW, 128), jnp.int32),
        mesh=mesh,
        compiler_params=pltpu.CompilerParams(needs_layout_passes=False),
        scratch_types=[
            pltpu.VMEM((_CHUNK,), jnp.float32),
            pltpu.VMEM((_CHUNK,), jnp.float32),
            pltpu.VMEM((16 * 128,), jnp.int32),
            pltpu.VMEM((48,), jnp.float32),
            pltpu.VMEM((128,), jnp.int32),
            pltpu.SemaphoreType.DMA,
            pltpu.SemaphoreType.DMA,
        ],
    )


def _sc_hist(data1d, scal48):
    return _get_sc_hist()(data1d, scal48)


def _transform_body(x_ref, hist_ref, scal_ref, par_ref, out_ref, tab_ref):
    i = pl.program_id(0)

    c = scal_ref[0]
    dmin = scal_ref[1]
    invw = scal_ref[2]
    spatio = scal_ref[3]

    @pl.when(i == 0)
    def _():
        hist_i = hist_ref[...]
        hcnt = jnp.sum(hist_i, axis=0, keepdims=True)
        r2 = lax.broadcasted_iota(jnp.int32, (128, 128), 0)
        l2 = lax.broadcasted_iota(jnp.int32, (128, 128), 1)
        hcnt_b = jnp.broadcast_to(hcnt, (128, 128))
        cum_col = jnp.sum(
            jnp.where((l2 <= r2) & (l2 < _POINTS), hcnt_b, 0),
            axis=1, keepdims=True)
        total = jnp.max(cum_col).astype(jnp.float32)
        accum_col = cum_col.astype(jnp.float32) * ((_POINTS - 1) / total)

        acc_b = jnp.broadcast_to(accum_col, (128, 128))
        acc_row = jnp.sum(jnp.where(r2 == l2, acc_b, 0.0),
                          axis=0, keepdims=True)
        acc_row8 = jnp.broadcast_to(acc_row, (8, 128))

        io_row8 = lax.broadcasted_iota(jnp.int32, (8, 128), 1)
        idxp1 = jnp.minimum(io_row8 + 1, _POINTS - 1)
        acc1_row8 = jnp.take_along_axis(acc_row8, idxp1, axis=1)

        tf = io_row8.astype(jnp.float32)
        ssum = jnp.sum(
            jnp.where((acc_b <= l2.astype(jnp.float32)) & (r2 < _POINTS), 1, 0),
            axis=0, keepdims=True)
        it = jnp.broadcast_to(jnp.clip(ssum, 1, _POINTS - 1), (8, 128))
        a1 = jnp.take_along_axis(acc_row8, it, axis=1)
        a0 = jnp.take_along_axis(acc_row8, it - 1, axis=1)
        dxv = a1 - a0
        dx0 = jnp.abs(dxv) <= _EPS_SP
        safe = jnp.where(dx0, 1.0, dxv)
        accum0 = jnp.sum(jnp.where((r2 == 0) & (l2 == 0), acc_b, 0.0))
        accum_last = jnp.max(accum_col)

        pt_row8 = jnp.broadcast_to(par_ref[0:1, :], (8, 128))
        pv_row8 = jnp.broadcast_to(par_ref[1:2, :], (8, 128))
        io1_row = lax.broadcasted_iota(jnp.int32, (1, 128), 1)
        pt0_s = jnp.sum(jnp.where(io1_row == 0, par_ref[0:1, :], 0.0))
        ptl_s = jnp.sum(jnp.where(io1_row == _POINTS - 1, par_ref[0:1, :], 0.0))
        pv0_s = jnp.sum(jnp.where(io1_row == 0, par_ref[1:2, :], 0.0))
        pvl_s = jnp.sum(jnp.where(io1_row == _POINTS - 1, par_ref[1:2, :], 0.0))

        def frame(p_row8, p0_s, pl_s):
            p1 = jnp.take_along_axis(p_row8, it, axis=1)
            p0 = jnp.take_along_axis(p_row8, it - 1, axis=1)
            f = jnp.where(dx0, p0, p0 + (tf - a0) / safe * (p1 - p0))
            f = jnp.where(tf < accum0, p0_s, f)
            f = jnp.where(tf > accum_last, pl_s, f)
            return f

        fth = frame(pt_row8, pt0_s, ptl_s)
        fve = frame(pv_row8, pv0_s, pvl_s)
        fth1 = jnp.take_along_axis(fth, idxp1, axis=1)
        fve1 = jnp.take_along_axis(fve, idxp1, axis=1)

        def tak8(tab, idx):
            return jnp.take_along_axis(tab, idx, axis=1,
                                       mode="promise_in_bounds")

        def eval_exact(dk):
            ugk = (dk - dmin) * invw - 0.5
            jk = jnp.clip(ugk.astype(jnp.int32), 0, _POINTS - 2)
            tgk = jnp.clip(ugk - jk.astype(jnp.float32), 0.0, 1.0)
            av0 = tak8(acc_row8, jk)
            av1 = tak8(acc1_row8, jk)
            idx = av0 + tgk * (av1 - av0)
            bk = jnp.minimum(idx.astype(jnp.int32), _POINTS - 1)
            posk = idx - bk.astype(jnp.float32)
            thk = ((1.0 - posk) * tak8(fth, bk) + posk * tak8(fth1, bk))
            vek = ((1.0 - posk) * tak8(fve, bk) + posk * tak8(fve1, bk))
            dsk = vek * 0.01
            return dsk * jnp.sin(thk), dsk * jnp.cos(thk)

        cw = (1.0 / invw) * (float(_POINTS) / _FINE)
        kf2 = lax.broadcasted_iota(jnp.int32, (8, 128), 1).astype(jnp.float32)
        dy0, dx0 = eval_exact(dmin + kf2 * cw)
        dy1, dx1 = eval_exact(dmin + (kf2 + 1.0) * cw)

        def pack(v0, v1):
            b0 = lax.bitcast_convert_type(v0, jnp.int32)
            b1 = lax.bitcast_convert_type(v1 - v0, jnp.int32)
            mask = jnp.int32(-65536)
            return (b0 & mask) | lax.shift_right_logical(b1, 16)

        tab_ref[0] = jnp.broadcast_to(pack(dy0, dy1)[0:1, :], (_RB, 128))
        tab_ref[1] = jnp.broadcast_to(pack(dx0, dx1)[0:1, :], (_RB, 128))

    def tak(tab, idx):
        return jnp.take_along_axis(tab, idx, axis=1, mode="promise_in_bounds")

    def unpack(g):
        mask = jnp.int32(-65536)
        v0 = lax.bitcast_convert_type(g & mask, jnp.float32)
        v1 = lax.bitcast_convert_type(lax.shift_left(g, 16), jnp.float32)
        return v0, v1

    x = x_ref[...]
    d = x * c
    u = (d - dmin) * (invw * (_FINE / float(_POINTS)))
    k = jnp.minimum(u.astype(jnp.int32), _FINE - 1)
    frac = u - k.astype(jnp.float32)
    dy0v, dysv = unpack(tak(tab_ref[0], k))
    dx0v, dxsv = unpack(tak(tab_ref[1], k))
    dyd = dy0v + frac * dysv
    dxd = dx0v + frac * dxsv
    out_ref[...] = (d * (1.0 + dyd) + dxd) * spatio


def _transform(x2d, hist, scal, par):
    return pl.pallas_call(
        _transform_body,
        grid=(_GRID,),
        in_specs=[
            pl.BlockSpec((_RB, _COLS), lambda i: (i, 0)),
            pl.BlockSpec((_NW, 128), lambda i: (0, 0)),
            pl.BlockSpec(memory_space=pltpu.SMEM),
            pl.BlockSpec((2, 128), lambda i: (0, 0)),
        ],
        out_specs=pl.BlockSpec((_RB, _COLS), lambda i: (i, 0)),
        out_shape=jax.ShapeDtypeStruct((_ROWS, _COLS), jnp.float32),
        scratch_shapes=[pltpu.VMEM((2, _RB, 128), jnp.int32)],
    )(x2d, hist, scal, par)


def kernel(data, params, channel_transform, spatio_transform):
    shape = data.shape
    x2d = data.reshape(_ROWS, _COLS)

    mm = _minmax(x2d)
    mn = mm[0, 0]
    mx = mm[1, 0]

    c = channel_transform.reshape(())
    spatio = spatio_transform.reshape(())
    dmin = jnp.where(c >= 0, c * mn, c * mx)
    dmax = jnp.where(c >= 0, c * mx, c * mn)
    width = (dmax - dmin) / _POINTS
    invw = 1.0 / width

    scal48 = jnp.concatenate([
        jnp.full((16,), c * invw, jnp.float32),
        jnp.full((16,), -dmin * invw, jnp.float32),
        jnp.full((16,), 0.0, jnp.float32),
    ])
    hist = _sc_hist(data.reshape(_N), scal48)

    scal4 = jnp.stack([c, dmin, invw, spatio]).astype(jnp.float32)
    par = jnp.pad(params.reshape(2, _POINTS), ((0, 0), (0, 128 - _POINTS)))
    out2d = _transform(x2d, hist, scal4, par)
    return out2d.reshape(shape)
